# baseline (speedup 1.0000x reference)
import functools

import jax
import jax.numpy as jnp
from jax import lax
from jax.experimental import pallas as pl
from jax.experimental.pallas import tpu as pltpu
from jax.experimental.pallas import tpu_sc as plsc

B = 4096
D = 128
H1 = 256
OUT = 128

_PREC = lax.Precision.DEFAULT


def _sc_gather_both(user_ids, item_ids, user_emb, item_emb, nb):
    info = plsc.get_sparse_core_info()
    nw = info.num_cores * info.num_subcores
    b_per_w = nb // nw

    mesh = plsc.VectorSubcoreMesh(core_axis_name="c", subcore_axis_name="s")

    @functools.partial(
        pl.kernel,
        mesh=mesh,
        out_type=[
            jax.ShapeDtypeStruct((nb, D), jnp.float32),
            jax.ShapeDtypeStruct((nb, D), jnp.float32),
        ],
        scratch_types=[
            pltpu.VMEM((b_per_w,), jnp.int32),
            pltpu.VMEM((b_per_w,), jnp.int32),
            pltpu.VMEM((b_per_w, D), jnp.float32),
            pltpu.VMEM((b_per_w, D), jnp.float32),
            pltpu.SemaphoreType.DMA,
            pltpu.SemaphoreType.DMA,
        ],
    )
    def gather2(uids_hbm, iids_hbm, uemb_hbm, iemb_hbm, out_u, out_i,
                uidx_v, iidx_v, urows_v, irows_v, usem, isem):
        wid = lax.axis_index("s") * info.num_cores + lax.axis_index("c")
        base = wid * b_per_w
        uic = pltpu.async_copy(uids_hbm.at[pl.ds(base, b_per_w)], uidx_v, usem)
        iic = pltpu.async_copy(iids_hbm.at[pl.ds(base, b_per_w)], iidx_v, isem)
        uic.wait()
        ucp = pltpu.async_copy(uemb_hbm.at[uidx_v], urows_v, usem)
        iic.wait()
        icp = pltpu.async_copy(iemb_hbm.at[iidx_v], irows_v, isem)
        ucp.wait()
        uoc = pltpu.async_copy(urows_v, out_u.at[pl.ds(base, b_per_w)], usem)
        icp.wait()
        ioc = pltpu.async_copy(irows_v, out_i.at[pl.ds(base, b_per_w)], isem)
        uoc.wait()
        ioc.wait()

    return gather2(user_ids, item_ids, user_emb, item_emb)


def _mm_body(a_ref, b_ref, o_ref):
    o_ref[...] = jnp.dot(a_ref[...], b_ref[...], precision=_PREC,
                         preferred_element_type=jnp.float32)


def _tc_matmul(a, b):
    return pl.pallas_call(
        _mm_body,
        out_shape=jax.ShapeDtypeStruct((a.shape[0], b.shape[1]), jnp.float32),
    )(a, b)


def _towers_body(eu_ref, ei_ref, M_ref, uW1_ref, ub1_ref, uW2_ref,
                 ub2_ref, iW1_ref, ib1_ref, iW2_ref, ib2_ref, u_ref, it_ref):
    def mm(a, b):
        return jnp.dot(a, b, precision=_PREC,
                       preferred_element_type=jnp.float32)

    eu = eu_ref[...]
    x = eu + mm(eu, M_ref[...])
    m = jnp.mean(x, axis=-1, keepdims=True)
    c = x - m
    v = jnp.mean(c * c, axis=-1, keepdims=True)
    h = c * lax.rsqrt(v + 1e-5)
    a = jnp.maximum(mm(h, uW1_ref[...]) + ub1_ref[...], 0.0)
    u = mm(a, uW2_ref[...]) + ub2_ref[...]
    u_ref[...] = u / (jnp.sqrt(jnp.sum(u * u, axis=-1, keepdims=True)) + 1e-12)

    ei = ei_ref[...]
    ai = jnp.maximum(mm(ei, iW1_ref[...]) + ib1_ref[...], 0.0)
    it = mm(ai, iW2_ref[...]) + ib2_ref[...]
    it_ref[...] = it / (jnp.sqrt(jnp.sum(it * it, axis=-1, keepdims=True))
                        + 1e-12)


def _tc_towers(eu, ei, M, uW1, ub1, uW2, ub2, iW1, ib1, iW2, ib2):
    nb = eu.shape[0]
    BT = 512
    grid = (nb // BT,)
    row_spec = pl.BlockSpec((BT, D), lambda i: (i, 0))
    out_spec = pl.BlockSpec((BT, OUT), lambda i: (i, 0))

    def w_spec(shape):
        return pl.BlockSpec(shape, lambda i: tuple(0 for _ in shape))

    return pl.pallas_call(
        _towers_body,
        grid=grid,
        in_specs=[
            row_spec, row_spec,
            w_spec((D, D)),
            w_spec((D, H1)), w_spec((1, H1)), w_spec((H1, OUT)),
            w_spec((1, OUT)),
            w_spec((D, H1)), w_spec((1, H1)), w_spec((H1, OUT)),
            w_spec((1, OUT)),
        ],
        out_specs=[out_spec, out_spec],
        out_shape=[
            jax.ShapeDtypeStruct((nb, OUT), jnp.float32),
            jax.ShapeDtypeStruct((nb, OUT), jnp.float32),
        ],
    )(eu, ei, M, uW1, ub1.reshape(1, H1), uW2, ub2.reshape(1, OUT),
      iW1, ib1.reshape(1, H1), iW2, ib2.reshape(1, OUT))


def kernel(user_ids, item_ids, user_emb, item_emb, Wq, Wk, Wv, Wo,
           uW1, ub1, uW2, ub2, iW1, ib1, iW2, ib2):
    del Wq, Wk
    uids = user_ids.astype(jnp.int32)
    iids = item_ids.astype(jnp.int32)
    M = _tc_matmul(Wv, Wo)
    eu, ei = _sc_gather_both(uids, iids, user_emb, item_emb, B)
    u, it = _tc_towers(eu, ei, M, uW1, ub1, uW2, ub2,
                       iW1, ib1, iW2, ib2)
    return (u, it)


# speedup vs baseline: 1.1366x; 1.1366x over previous
import functools

import jax
import jax.numpy as jnp
from jax import lax
from jax.experimental import pallas as pl
from jax.experimental.pallas import tpu as pltpu
from jax.experimental.pallas import tpu_sc as plsc

B = 4096
D = 128
H1 = 256
OUT = 128

_PREC = lax.Precision.DEFAULT


def _sc_gather_both(user_ids, item_ids, user_emb, item_emb, nb):
    info = plsc.get_sparse_core_info()
    nw = info.num_cores * info.num_subcores
    b_per_w = nb // nw

    mesh = plsc.VectorSubcoreMesh(core_axis_name="c", subcore_axis_name="s")

    @functools.partial(
        pl.kernel,
        mesh=mesh,
        out_type=[
            jax.ShapeDtypeStruct((nb, D), jnp.float32),
            jax.ShapeDtypeStruct((nb, D), jnp.float32),
        ],
        scratch_types=[
            pltpu.VMEM((b_per_w,), jnp.int32),
            pltpu.VMEM((b_per_w,), jnp.int32),
            pltpu.VMEM((b_per_w, D), jnp.float32),
            pltpu.VMEM((b_per_w, D), jnp.float32),
            pltpu.SemaphoreType.DMA,
            pltpu.SemaphoreType.DMA,
        ],
    )
    def gather2(uids_hbm, iids_hbm, uemb_hbm, iemb_hbm, out_u, out_i,
                uidx_v, iidx_v, urows_v, irows_v, usem, isem):
        wid = lax.axis_index("s") * info.num_cores + lax.axis_index("c")
        base = wid * b_per_w
        uic = pltpu.async_copy(uids_hbm.at[pl.ds(base, b_per_w)], uidx_v, usem)
        iic = pltpu.async_copy(iids_hbm.at[pl.ds(base, b_per_w)], iidx_v, isem)
        uic.wait()
        ucp = pltpu.async_copy(uemb_hbm.at[uidx_v], urows_v, usem)
        iic.wait()
        icp = pltpu.async_copy(iemb_hbm.at[iidx_v], irows_v, isem)
        ucp.wait()
        uoc = pltpu.async_copy(urows_v, out_u.at[pl.ds(base, b_per_w)], usem)
        icp.wait()
        ioc = pltpu.async_copy(irows_v, out_i.at[pl.ds(base, b_per_w)], isem)
        uoc.wait()
        ioc.wait()

    return gather2(user_ids, item_ids, user_emb, item_emb)


def _mm_body(a_ref, b_ref, o_ref):
    o_ref[...] = jnp.dot(a_ref[...], b_ref[...], precision=_PREC,
                         preferred_element_type=jnp.float32)


def _tc_matmul(a, b):
    return pl.pallas_call(
        _mm_body,
        out_shape=jax.ShapeDtypeStruct((a.shape[0], b.shape[1]), jnp.float32),
    )(a, b)


def _towers_body(eu_ref, ei_ref, M_ref, uW1_ref, ub1_ref, uW2_ref,
                 ub2_ref, iW1_ref, ib1_ref, iW2_ref, ib2_ref, u_ref, it_ref):
    def mm(a, b):
        return jnp.dot(a, b, precision=_PREC,
                       preferred_element_type=jnp.float32)

    eu = eu_ref[...]
    x = eu + mm(eu, M_ref[...])
    m = jnp.mean(x, axis=-1, keepdims=True)
    c = x - m
    v = jnp.mean(c * c, axis=-1, keepdims=True)
    h = c * lax.rsqrt(v + 1e-5)
    a = jnp.maximum(mm(h, uW1_ref[...]) + ub1_ref[...], 0.0)
    u = mm(a, uW2_ref[...]) + ub2_ref[...]
    u_ref[...] = u / (jnp.sqrt(jnp.sum(u * u, axis=-1, keepdims=True)) + 1e-12)

    ei = ei_ref[...]
    ai = jnp.maximum(mm(ei, iW1_ref[...]) + ib1_ref[...], 0.0)
    it = mm(ai, iW2_ref[...]) + ib2_ref[...]
    it_ref[...] = it / (jnp.sqrt(jnp.sum(it * it, axis=-1, keepdims=True))
                        + 1e-12)


def _tc_towers(eu, ei, M, uW1, ub1, uW2, ub2, iW1, ib1, iW2, ib2):
    nb = eu.shape[0]
    BT = 2048
    grid = (nb // BT,)
    row_spec = pl.BlockSpec((BT, D), lambda i: (i, 0))
    out_spec = pl.BlockSpec((BT, OUT), lambda i: (i, 0))

    def w_spec(shape):
        return pl.BlockSpec(shape, lambda i: tuple(0 for _ in shape))

    return pl.pallas_call(
        _towers_body,
        grid=grid,
        in_specs=[
            row_spec, row_spec,
            w_spec((D, D)),
            w_spec((D, H1)), w_spec((1, H1)), w_spec((H1, OUT)),
            w_spec((1, OUT)),
            w_spec((D, H1)), w_spec((1, H1)), w_spec((H1, OUT)),
            w_spec((1, OUT)),
        ],
        out_specs=[out_spec, out_spec],
        out_shape=[
            jax.ShapeDtypeStruct((nb, OUT), jnp.float32),
            jax.ShapeDtypeStruct((nb, OUT), jnp.float32),
        ],
    )(eu, ei, M, uW1, ub1.reshape(1, H1), uW2, ub2.reshape(1, OUT),
      iW1, ib1.reshape(1, H1), iW2, ib2.reshape(1, OUT))


def kernel(user_ids, item_ids, user_emb, item_emb, Wq, Wk, Wv, Wo,
           uW1, ub1, uW2, ub2, iW1, ib1, iW2, ib2):
    del Wq, Wk
    uids = user_ids.astype(jnp.int32)
    iids = item_ids.astype(jnp.int32)
    M = _tc_matmul(Wv, Wo)
    eu, ei = _sc_gather_both(uids, iids, user_emb, item_emb, B)
    u, it = _tc_towers(eu, ei, M, uW1, ub1, uW2, ub2,
                       iW1, ib1, iW2, ib2)
    return (u, it)
